# baseline (speedup 1.0000x reference)
import functools

import jax
import jax.numpy as jnp
from jax import lax
from jax.experimental import pallas as pl
from jax.experimental.pallas import tpu as pltpu
from jax.experimental.pallas import tpu_sc as plsc

B = 4
T = 8192
D = 1024
NC = 2
NS = 16
NW = NC * NS
TPW = T // NW
K = 32
NCHUNK = TPW // K
NSL = D // 16


@functools.partial(
    pl.kernel,
    out_type=jax.ShapeDtypeStruct((B * T, D), jnp.float32),
    mesh=plsc.VectorSubcoreMesh(core_axis_name="c", subcore_axis_name="s"),
    scratch_types=[
        pltpu.VMEM((K,), jnp.int32),
        pltpu.VMEM((K, D), jnp.float32),
        pltpu.VMEM((K, D), jnp.float32),
        pltpu.SemaphoreType.DMA,
    ],
)
def _emb_kernel(x_hbm, tok_hbm, pos_hbm, out_hbm, idx_v, rows_v, pos_v, sem):
    wid = lax.axis_index("s") * NC + lax.axis_index("c")
    t0 = wid * TPW

    def chunk_body(ci, carry):
        tb = t0 + ci * K
        pltpu.sync_copy(pos_hbm.at[pl.ds(tb, K)], pos_v)
        for b in range(B):
            pltpu.sync_copy(x_hbm.at[pl.ds(b * T + tb, K)], idx_v)
            pltpu.async_copy(tok_hbm.at[idx_v], rows_v, sem).wait()

            def add_row(r, c2):
                for j in range(NSL):
                    sl = pl.ds(j * 16, 16)
                    plsc.addupdate(rows_v.at[r, sl], pos_v[r, sl])
                return c2

            lax.fori_loop(0, K, add_row, 0)
            pltpu.sync_copy(rows_v, out_hbm.at[pl.ds(b * T + tb, K)])
        return carry

    lax.fori_loop(0, NCHUNK, chunk_body, 0)


def kernel(x, tok_emb, pos_emb):
    out = _emb_kernel(x.reshape(B * T), tok_emb, pos_emb)
    return out.reshape(B, T, D)


# speedup vs baseline: 1.8796x; 1.8796x over previous
import functools

import jax
import jax.numpy as jnp
from jax import lax
from jax.experimental import pallas as pl
from jax.experimental.pallas import tpu as pltpu
from jax.experimental.pallas import tpu_sc as plsc

B = 4
T = 8192
D = 1024
NC = 2
NS = 16
NW = NC * NS
TPW = T // NW
K = 8
NCHUNK = TPW // K
NSL = D // 16

_SCRATCH = (
    [pltpu.VMEM((B, TPW), jnp.int32)]
    + [pltpu.VMEM((K, D), jnp.float32) for _ in range(8)]
    + [pltpu.VMEM((K, D), jnp.float32) for _ in range(2)]
    + [pltpu.SemaphoreType.DMA for _ in range(8)]
    + [pltpu.SemaphoreType.DMA for _ in range(8)]
    + [pltpu.SemaphoreType.DMA for _ in range(2)]
)


@functools.partial(
    pl.kernel,
    out_type=jax.ShapeDtypeStruct((B * T, D), jnp.float32),
    mesh=plsc.VectorSubcoreMesh(core_axis_name="c", subcore_axis_name="s"),
    scratch_types=_SCRATCH,
)
def _emb_kernel(x_hbm, tok_hbm, pos_hbm, out_hbm, idx_v, *bufs):
    rbuf = [[bufs[p * B + b] for b in range(B)] for p in range(2)]
    pos_v = [bufs[8], bufs[9]]
    gsem = [[bufs[10 + p * B + b] for b in range(B)] for p in range(2)]
    ssem = [[bufs[18 + p * B + b] for b in range(B)] for p in range(2)]
    psem = [bufs[26], bufs[27]]

    wid = lax.axis_index("s") * NC + lax.axis_index("c")
    t0 = wid * TPW

    for b in range(B):
        pltpu.sync_copy(x_hbm.at[pl.ds(b * T + t0, TPW)], idx_v.at[b])

    pltpu.async_copy(pos_hbm.at[pl.ds(t0, K)], pos_v[0], psem[0])
    for b in range(B):
        pltpu.async_copy(tok_hbm.at[idx_v.at[b, pl.ds(0, K)]], rbuf[0][b],
                         gsem[0][b])

    def half(h, carry):
        for p in range(2):
            ci = 2 * h + p
            tb = t0 + ci * K
            nxt = ci + 1 < NCHUNK

            pltpu.make_async_copy(pos_hbm.at[pl.ds(tb, K)], pos_v[p],
                                  psem[p]).wait()

            @pl.when(nxt)
            def _():
                pltpu.async_copy(pos_hbm.at[pl.ds(tb + K, K)], pos_v[1 - p],
                                 psem[1 - p])

            for b in range(B):
                o = b * T + tb
                pltpu.make_async_copy(tok_hbm.at[idx_v.at[b, pl.ds(ci * K, K)]],
                                      rbuf[p][b], gsem[p][b]).wait()

                def add_row(r, c2):
                    for j in range(NSL):
                        sl = pl.ds(j * 16, 16)
                        plsc.addupdate(rbuf[p][b].at[r, sl], pos_v[p][r, sl])
                    return c2

                lax.fori_loop(0, K, add_row, 0)
                pltpu.async_copy(rbuf[p][b], out_hbm.at[pl.ds(o, K)],
                                 ssem[p][b])

                @pl.when(jnp.logical_and(ci >= 1, nxt))
                def _():
                    pltpu.make_async_copy(rbuf[1 - p][b],
                                          out_hbm.at[pl.ds(o, K)],
                                          ssem[1 - p][b]).wait()

                @pl.when(nxt)
                def _():
                    pltpu.async_copy(
                        tok_hbm.at[idx_v.at[b, pl.ds((ci + 1) * K, K)]],
                        rbuf[1 - p][b], gsem[1 - p][b])
        return carry

    lax.fori_loop(0, NCHUNK // 2, half, 0)

    for p in range(2):
        for b in range(B):
            pltpu.make_async_copy(rbuf[p][b], out_hbm.at[pl.ds(t0, K)],
                                  ssem[p][b]).wait()


def kernel(x, tok_emb, pos_emb):
    out = _emb_kernel(x.reshape(B * T), tok_emb, pos_emb)
    return out.reshape(B, T, D)
